# baseline (speedup 1.0000x reference)
import functools

import jax
import jax.numpy as jnp
from jax import lax
from jax.experimental import pallas as pl
from jax.experimental.pallas import tpu as pltpu
from jax.experimental.pallas import tpu_sc as plsc

_SIZES = {"assmpt": 34000, "non_assmpt": 33000, "rule": 33000}
_RELS = [("assmpt", "rule"), ("non_assmpt", "rule"), ("non_assmpt", "assmpt"),
         ("assmpt", "assmpt"), ("rule", "non_assmpt"), ("rule", "assmpt"),
         ("assmpt", "assmpt"), ("rule", "rule"), ("non_assmpt", "non_assmpt")]
_TYPES = ("assmpt", "non_assmpt", "rule")
_OUT_RELS = {t: [i for i, (s, _) in enumerate(_RELS) if s == t] for t in _TYPES}
_IN_RELS = {t: [i for i, (_, d) in enumerate(_RELS) if d == t] for t in _TYPES}
_BLK = 256

_E = 64000
_R = 12288
_NB = 3
_NPAD = _NB * _R
_ACC = _R + 8
_K = 16
_KSH = 4
_D = 12
_CAP = 2112
_NT = 32
_EPT = _E // _NT
_DS = 34048
_DEGN = 9 * _DS
_DEGT = _DEGN // 16
_mesh = plsc.VectorSubcoreMesh(core_axis_name="c", subcore_axis_name="s")
_f32 = jnp.float32
_i32 = jnp.int32


def _prep_body(s0, s1, s2, s3, s4, s5, s6, s7, s8,
               d0, d1, d2, d3, d4, d5, d6, d7, d8,
               gsrc, gdst, cnt, dego, degi,
               sstage, dstage, bs0, bs1, bs2, bd0, bd1, bd2,
               cntloc, ones_v, dsst, ddst, zbuf, dego_s, degi_s):
    bsrcs, bdsts = (bs0, bs1, bs2), (bd0, bd1, bd2)
    esrc = (s0, s1, s2, s3, s4, s5, s6, s7, s8)
    edst = (d0, d1, d2, d3, d4, d5, d6, d7, d8)
    c = lax.axis_index("c")
    s = lax.axis_index("s")
    wid = 2 * s + c

    def fill(i, _):
        zbuf[pl.ds(i * 16, 16)] = jnp.zeros((16,), _f32)
        return 0
    lax.fori_loop(0, _DEGT // 16, fill, 0)

    def fillo(i, _):
        ones_v[pl.ds(i * 16, 16)] = jnp.ones((16,), _f32)
        return 0
    lax.fori_loop(0, 250, fillo, 0)

    pltpu.sync_copy(zbuf, dego_s.at[pl.ds(s * _DEGT, _DEGT)])
    pltpu.sync_copy(zbuf, degi_s.at[pl.ds(s * _DEGT, _DEGT)])
    plsc.subcore_barrier()

    for r in range(9):
        pltpu.sync_copy(esrc[r].at[pl.ds(wid * _EPT, _EPT)], sstage)
        pltpu.sync_copy(edst[r].at[pl.ds(wid * _EPT, _EPT)], dstage)

        def it(i, offs):
            s16 = sstage[pl.ds(i * 16, 16)]
            d16 = dstage[pl.ds(i * 16, 16)]
            dump = _CAP + lax.iota(_i32, 16)
            new = []
            for b in range(_NB):
                lo = b * _R
                m = d16 < lo + _R if b == 0 else (d16 >= lo) & (d16 < lo + _R)
                mi = jnp.where(m, 1, 0).astype(_i32)
                incl = jnp.cumsum(mi)
                pos = jnp.where(m, offs[b] + incl - mi, dump)
                plsc.store_scatter(bsrcs[b], [pos], s16)
                plsc.store_scatter(bdsts[b], [pos], d16 - lo)
                new.append(offs[b] + jnp.max(plsc.all_reduce_population_count(m)))
            return tuple(new)

        offs = lax.fori_loop(0, _EPT // 16, it, (_i32(0), _i32(0), _i32(0)))
        for b in range(_NB):
            for j in range(4):
                bsrcs[b][pl.ds(offs[b] + j * 16, 16)] = jnp.zeros((16,), _i32)
                bdsts[b][pl.ds(offs[b] + j * 16, 16)] = jnp.full((16,), _R, _i32)
            lane = r * _NB + b
            base = (lane // 16) * 16
            old = cntloc[pl.ds(base, 16)]
            cntloc[pl.ds(base, 16)] = jnp.where(
                lax.iota(_i32, 16) == (lane % 16), offs[b], old)
        for b in range(_NB):
            base = ((r * _NT + wid) * _NB + b) * _CAP
            pltpu.sync_copy(bsrcs[b].at[pl.ds(0, _CAP)], gsrc.at[pl.ds(base, _CAP)])
            pltpu.sync_copy(bdsts[b].at[pl.ds(0, _CAP)], gdst.at[pl.ds(base, _CAP)])

        @pl.when(c == r % 2)
        def _deg():
            pltpu.sync_copy(esrc[r].at[pl.ds(s * 4000, 4000)], dsst)
            pltpu.sync_copy(edst[r].at[pl.ds(s * 4000, 4000)], ddst)

            def off(i, _):
                dsst[pl.ds(i * 16, 16)] = dsst[pl.ds(i * 16, 16)] + r * _DS
                ddst[pl.ds(i * 16, 16)] = ddst[pl.ds(i * 16, 16)] + r * _DS
                return 0
            lax.fori_loop(0, 250, off, 0)
            pltpu.sync_copy(ones_v, dego_s.at[dsst], add=True)
            pltpu.sync_copy(ones_v, degi_s.at[ddst], add=True)

    plsc.subcore_barrier()
    pltpu.sync_copy(cntloc, cnt.at[pl.ds(wid * 48, 48)])
    pltpu.sync_copy(dego_s.at[pl.ds(s * _DEGT, _DEGT)], zbuf)
    pltpu.sync_copy(zbuf, dego.at[pl.ds(c * _DEGN + s * _DEGT, _DEGT)])
    pltpu.sync_copy(degi_s.at[pl.ds(s * _DEGT, _DEGT)], zbuf)
    pltpu.sync_copy(zbuf, degi.at[pl.ds(c * _DEGN + s * _DEGT, _DEGT)])


_prep_call = functools.partial(
    pl.kernel, _prep_body, mesh=_mesh,
    compiler_params=pltpu.CompilerParams(needs_layout_passes=False),
    out_type=[jax.ShapeDtypeStruct((9 * _NT * _NB * _CAP,), _i32),
              jax.ShapeDtypeStruct((9 * _NT * _NB * _CAP,), _i32),
              jax.ShapeDtypeStruct((_NT * 48,), _i32),
              jax.ShapeDtypeStruct((2 * _DEGN,), _f32),
              jax.ShapeDtypeStruct((2 * _DEGN,), _f32)],
    scratch_types=[pltpu.VMEM((_EPT,), _i32),
                   pltpu.VMEM((_EPT,), _i32)]
    + [pltpu.VMEM((_CAP + 16,), _i32)] * 6
    + [pltpu.VMEM((48,), _i32),
                   pltpu.VMEM((4000,), _f32),
                   pltpu.VMEM((4000,), _i32),
                   pltpu.VMEM((4000,), _i32),
                   pltpu.VMEM((_DEGT,), _f32),
                   pltpu.VMEM_SHARED((_DEGN,), _f32),
                   pltpu.VMEM_SHARED((_DEGN,), _f32)],
)()


def _segsum_body(rels, refs):
    n_z = len(rels)
    zs = refs[:n_z]
    (gsrc, gdst, cnt) = refs[n_z:n_z + 3]
    aggs = refs[n_z + 3:2 * n_z + 3]
    rest = refs[2 * n_z + 3:]
    acc, sseg, dseg = rest[0], rest[1], rest[2]
    rbufs = rest[3:3 + _D]
    zrow, cntloc = rest[3 + _D], rest[4 + _D]
    sems = rest[5 + _D:5 + 2 * _D]
    c = lax.axis_index("c")
    s = lax.axis_index("s")

    def fill(i, _):
        def fl(j, _):
            zrow[i, pl.ds(j * 16, 16)] = jnp.zeros((16,), _f32)
            return 0
        lax.fori_loop(0, 8, fl, 0)
        return 0
    lax.fori_loop(0, 16, fill, 0)
    pltpu.sync_copy(cnt.at[pl.ds(2 * s * 48, 96)], cntloc)

    for task in range(n_z * _NB):
        ri, b = task // _NB, task % _NB
        r = rels[ri]
        owner = task % 2
        zref = zs[ri]
        aref = aggs[ri]

        @pl.when(c == owner)
        def _task():
            for j in range(48):
                pltpu.sync_copy(zrow, acc.at[pl.ds(s * 768 + j * 16, 16)])
            plsc.subcore_barrier()
            for seg in range(2):
                t = 2 * s + seg
                gbase = ((r * _NT + t) * _NB + b) * _CAP
                pltpu.sync_copy(gsrc.at[pl.ds(gbase, _CAP)], sseg)
                pltpu.sync_copy(gdst.at[pl.ds(gbase, _CAP)], dseg)
                lane = r * _NB + b
                vec = cntloc[pl.ds(seg * 48 + (lane // 16) * 16, 16)]
                iot = lax.iota(_i32, 16)
                cval = jnp.max(jnp.where(iot == (lane % 16), vec, 0))
                nch = (cval + (_K - 1)) >> _KSH

                for p in range(_D - 1):
                    @pl.when(p < nch)
                    def _prime():
                        pltpu.async_copy(zref.at[sseg.at[pl.ds(p * _K, _K)]],
                                         rbufs[p], sems[p])

                def chunk(ci, _):
                    for par in range(_D):
                        rb, sm = rbufs[par], sems[par]

                        @pl.when(ci % _D == par)
                        def _():
                            nxt = ci + (_D - 1)

                            @pl.when(nxt < nch)
                            def _start_next():
                                pltpu.async_copy(
                                    zref.at[sseg.at[pl.ds(nxt * _K, _K)]],
                                    rbufs[(par + _D - 1) % _D],
                                    sems[(par + _D - 1) % _D])
                            pltpu.make_async_copy(
                                zref.at[pl.ds(0, _K)], rb, sm).wait()
                            pltpu.sync_copy(
                                rb, acc.at[dseg.at[pl.ds(ci * _K, _K)]],
                                add=True)
                    return 0
                lax.fori_loop(0, nch, chunk, 0)
            plsc.subcore_barrier()
            pltpu.sync_copy(acc.at[pl.ds(s * 768, 768)],
                            aref.at[pl.ds(b * _R + s * 768, 768)])


def _segsum_call(rels, zs, gsrc, gdst, cnt):
    def body(*refs):
        _segsum_body(rels, refs)

    call = functools.partial(
        pl.kernel, body, mesh=_mesh,
        compiler_params=pltpu.CompilerParams(needs_layout_passes=False),
        out_type=[jax.ShapeDtypeStruct((_NPAD, 128), _f32)] * len(rels),
        scratch_types=[pltpu.VMEM_SHARED((_ACC, 128), _f32),
                       pltpu.VMEM((_CAP,), _i32),
                       pltpu.VMEM((_CAP,), _i32)]
        + [pltpu.VMEM((_K, 128), _f32)] * _D
        + [pltpu.VMEM((16, 128), _f32),
           pltpu.VMEM((96,), _i32)]
        + [pltpu.SemaphoreType.DMA] * _D,
    )()
    return call(*zs, gsrc, gdst, cnt)


def _scale(deg_blk):
    return jax.lax.rsqrt(jnp.maximum(deg_blk, 1.0))


def _stage_out_body(x_ref, d0, d1, d2, w0, w1, w2, z0, z1, z2):
    x = x_ref[...]
    for dref, wref, zref in ((d0, w0, z0), (d1, w1, z1), (d2, w2, z2)):
        s = _scale(dref[...])
        zref[...] = jnp.dot(x * s[:, None], wref[...],
                            preferred_element_type=jnp.float32)


def _stage_out(x, degs, Ws):
    n = x.shape[0]
    nblk = pl.cdiv(n, _BLK)
    row = pl.BlockSpec((_BLK, 128), lambda j: (j, 0))
    vec = pl.BlockSpec((_BLK,), lambda j: (j,))
    wsp = pl.BlockSpec((128, 128), lambda j: (0, 0))
    return pl.pallas_call(
        _stage_out_body,
        grid=(nblk,),
        in_specs=[row, vec, vec, vec, wsp, wsp, wsp],
        out_specs=[row, row, row],
        out_shape=[jax.ShapeDtypeStruct((n, 128), jnp.float32)] * 3,
    )(x, degs[0], degs[1], degs[2], Ws[0], Ws[1], Ws[2])


def _stage_mid_body(n_in, refs):
    aggs = refs[:n_in]
    dins = refs[n_in:2 * n_in]
    bsum = refs[2 * n_in]
    douts = refs[2 * n_in + 1:2 * n_in + 4]
    ws = refs[2 * n_in + 4:2 * n_in + 7]
    zs = refs[2 * n_in + 7:]
    h = bsum[...]
    for a, d in zip(aggs, dins):
        h = h + a[...] * _scale(d[...])[:, None]
    h = jnp.maximum(h, 0.0)
    for d, w, z in zip(douts, ws, zs):
        s = _scale(d[...])
        z[...] = jnp.dot(h * s[:, None], w[...],
                         preferred_element_type=jnp.float32)


def _stage_mid(aggs, dins, bsum, douts, Ws, n):
    n_in = len(aggs)
    nblk = pl.cdiv(n, _BLK)
    row = pl.BlockSpec((_BLK, 128), lambda j: (j, 0))
    vec = pl.BlockSpec((_BLK,), lambda j: (j,))
    wsp = pl.BlockSpec((128, 128), lambda j: (0, 0))
    bsp = pl.BlockSpec((1, 128), lambda j: (0, 0))

    def body(*refs):
        _stage_mid_body(n_in, refs)

    return pl.pallas_call(
        body,
        grid=(nblk,),
        in_specs=[row] * n_in + [vec] * n_in + [bsp] + [vec] * 3 + [wsp] * 3,
        out_specs=[row] * 3,
        out_shape=[jax.ShapeDtypeStruct((n, 128), jnp.float32)] * 3,
    )(*aggs, *dins, bsum.reshape(1, 128), *douts, *Ws)


def _stage_final_body(n_in, refs):
    aggs = refs[:n_in]
    dins = refs[n_in:2 * n_in]
    bsum = refs[2 * n_in]
    wc = refs[2 * n_in + 1]
    bc = refs[2 * n_in + 2]
    out = refs[2 * n_in + 3]
    h = bsum[...]
    for a, d in zip(aggs, dins):
        h = h + a[...] * _scale(d[...])[:, None]
    h = jnp.maximum(h, 0.0)
    out[...] = jnp.dot(h, wc[...], preferred_element_type=jnp.float32) + bc[...]


def _stage_final(aggs, dins, bsum, Wc, bc, n):
    n_in = len(aggs)
    nblk = pl.cdiv(n, _BLK)
    row = pl.BlockSpec((_BLK, 128), lambda j: (j, 0))
    vec = pl.BlockSpec((_BLK,), lambda j: (j,))
    bsp = pl.BlockSpec((1, 128), lambda j: (0, 0))

    def body(*refs):
        _stage_final_body(n_in, refs)

    return pl.pallas_call(
        body,
        grid=(nblk,),
        in_specs=[row] * n_in + [vec] * n_in + [bsp]
        + [pl.BlockSpec((128, 16), lambda j: (0, 0)),
           pl.BlockSpec((1, 16), lambda j: (0, 0))],
        out_specs=pl.BlockSpec((_BLK, 16), lambda j: (j, 0)),
        out_shape=jax.ShapeDtypeStruct((n, 16), jnp.float32),
    )(*aggs, *dins, bsum.reshape(1, 128), Wc, bc.reshape(1, 16))


def kernel(x_assmpt, x_non_assmpt, x_rule, e0, e1, e2, e3, e4, e5, e6, e7, e8,
           W1, b1, W2, b2, Wc, bc):
    xs = {"assmpt": x_assmpt, "non_assmpt": x_non_assmpt, "rule": x_rule}
    edges = [e.astype(jnp.int32) for e in (e0, e1, e2, e3, e4, e5, e6, e7, e8)]

    gsrc, gdst, cnt, dego, degi = _prep_call(
        *[e[0] for e in edges], *[e[1] for e in edges])
    dego = dego.reshape(2, _DEGN)
    degi = degi.reshape(2, _DEGN)
    deg_out = [dego[i % 2, i * _DS:i * _DS + _SIZES[s]]
               for i, (s, _) in enumerate(_RELS)]
    deg_in = [degi[i % 2, i * _DS:i * _DS + _SIZES[d]]
              for i, (_, d) in enumerate(_RELS)]
    bsum1 = {t: functools.reduce(jnp.add, (b1[i] for i in _IN_RELS[t])) for t in _TYPES}
    bsum2 = {t: functools.reduce(jnp.add, (b2[i] for i in _IN_RELS[t])) for t in _TYPES}

    z1 = [None] * 9
    for t in _TYPES:
        rels = _OUT_RELS[t]
        zt = _stage_out(xs[t], [deg_out[i] for i in rels], [W1[i] for i in rels])
        for i, z in zip(rels, zt):
            z1[i] = z
    agg1 = {t: _segsum_call(tuple(_IN_RELS[t]), [z1[i] for i in _IN_RELS[t]],
                            gsrc, gdst, cnt) for t in _TYPES}

    z2 = [None] * 9
    for t in _TYPES:
        irels = _IN_RELS[t]
        orels = _OUT_RELS[t]
        zt = _stage_mid(agg1[t], [deg_in[i] for i in irels],
                        bsum1[t], [deg_out[j] for j in orels],
                        [W2[j] for j in orels], _SIZES[t])
        for j, z in zip(orels, zt):
            z2[j] = z
    agg2 = {t: _segsum_call(tuple(_IN_RELS[t]), [z2[i] for i in _IN_RELS[t]],
                            gsrc, gdst, cnt) for t in _TYPES}

    outs = {}
    for t in _TYPES:
        irels = _IN_RELS[t]
        outs[t] = _stage_final(agg2[t],
                               [deg_in[i] for i in irels], bsum2[t], Wc, bc,
                               _SIZES[t])
    return (outs["assmpt"], outs["non_assmpt"], outs["rule"])


# speedup vs baseline: 1.0736x; 1.0736x over previous
import functools

import jax
import jax.numpy as jnp
from jax import lax
from jax.experimental import pallas as pl
from jax.experimental.pallas import tpu as pltpu
from jax.experimental.pallas import tpu_sc as plsc

_SIZES = {"assmpt": 34000, "non_assmpt": 33000, "rule": 33000}
_RELS = [("assmpt", "rule"), ("non_assmpt", "rule"), ("non_assmpt", "assmpt"),
         ("assmpt", "assmpt"), ("rule", "non_assmpt"), ("rule", "assmpt"),
         ("assmpt", "assmpt"), ("rule", "rule"), ("non_assmpt", "non_assmpt")]
_TYPES = ("assmpt", "non_assmpt", "rule")
_OUT_RELS = {t: [i for i, (s, _) in enumerate(_RELS) if s == t] for t in _TYPES}
_IN_RELS = {t: [i for i, (_, d) in enumerate(_RELS) if d == t] for t in _TYPES}
_BLK = 256

_E = 64000
_R = 12288
_NB = 3
_NPAD = _NB * _R
_ACC = _R + 8
_K = 8
_KSH = 3
_D = 16
_CAP = 2112
_NT = 32
_EPT = _E // _NT
_DS = 34048
_DEGN = 9 * _DS
_DEGT = _DEGN // 16
_mesh = plsc.VectorSubcoreMesh(core_axis_name="c", subcore_axis_name="s")
_f32 = jnp.float32
_i32 = jnp.int32


def _prep_body(s0, s1, s2, s3, s4, s5, s6, s7, s8,
               d0, d1, d2, d3, d4, d5, d6, d7, d8,
               gsrc, gdst, cnt, dego, degi,
               sstage, dstage, bs0, bs1, bs2, bd0, bd1, bd2,
               cntloc, ones_v, dsst, ddst, zbuf, dego_s, degi_s):
    bsrcs, bdsts = (bs0, bs1, bs2), (bd0, bd1, bd2)
    esrc = (s0, s1, s2, s3, s4, s5, s6, s7, s8)
    edst = (d0, d1, d2, d3, d4, d5, d6, d7, d8)
    c = lax.axis_index("c")
    s = lax.axis_index("s")
    wid = 2 * s + c

    def fill(i, _):
        zbuf[pl.ds(i * 16, 16)] = jnp.zeros((16,), _f32)
        return 0
    lax.fori_loop(0, _DEGT // 16, fill, 0)

    def fillo(i, _):
        ones_v[pl.ds(i * 16, 16)] = jnp.ones((16,), _f32)
        return 0
    lax.fori_loop(0, 250, fillo, 0)

    pltpu.sync_copy(zbuf, dego_s.at[pl.ds(s * _DEGT, _DEGT)])
    pltpu.sync_copy(zbuf, degi_s.at[pl.ds(s * _DEGT, _DEGT)])
    plsc.subcore_barrier()

    for r in range(9):
        pltpu.sync_copy(esrc[r].at[pl.ds(wid * _EPT, _EPT)], sstage)
        pltpu.sync_copy(edst[r].at[pl.ds(wid * _EPT, _EPT)], dstage)

        def it(i, offs):
            s16 = sstage[pl.ds(i * 16, 16)]
            d16 = dstage[pl.ds(i * 16, 16)]
            dump = _CAP + lax.iota(_i32, 16)
            new = []
            for b in range(_NB):
                lo = b * _R
                m = d16 < lo + _R if b == 0 else (d16 >= lo) & (d16 < lo + _R)
                mi = jnp.where(m, 1, 0).astype(_i32)
                incl = jnp.cumsum(mi)
                pos = jnp.where(m, offs[b] + incl - mi, dump)
                plsc.store_scatter(bsrcs[b], [pos], s16)
                plsc.store_scatter(bdsts[b], [pos], d16 - lo)
                new.append(offs[b] + jnp.max(plsc.all_reduce_population_count(m)))
            return tuple(new)

        offs = lax.fori_loop(0, _EPT // 16, it, (_i32(0), _i32(0), _i32(0)))
        for b in range(_NB):
            for j in range(4):
                bsrcs[b][pl.ds(offs[b] + j * 16, 16)] = jnp.zeros((16,), _i32)
                bdsts[b][pl.ds(offs[b] + j * 16, 16)] = jnp.full((16,), _R, _i32)
            lane = r * _NB + b
            base = (lane // 16) * 16
            old = cntloc[pl.ds(base, 16)]
            cntloc[pl.ds(base, 16)] = jnp.where(
                lax.iota(_i32, 16) == (lane % 16), offs[b], old)
        for b in range(_NB):
            base = ((r * _NT + wid) * _NB + b) * _CAP
            pltpu.sync_copy(bsrcs[b].at[pl.ds(0, _CAP)], gsrc.at[pl.ds(base, _CAP)])
            pltpu.sync_copy(bdsts[b].at[pl.ds(0, _CAP)], gdst.at[pl.ds(base, _CAP)])

        @pl.when(c == r % 2)
        def _deg():
            pltpu.sync_copy(esrc[r].at[pl.ds(s * 4000, 4000)], dsst)
            pltpu.sync_copy(edst[r].at[pl.ds(s * 4000, 4000)], ddst)

            def off(i, _):
                dsst[pl.ds(i * 16, 16)] = dsst[pl.ds(i * 16, 16)] + r * _DS
                ddst[pl.ds(i * 16, 16)] = ddst[pl.ds(i * 16, 16)] + r * _DS
                return 0
            lax.fori_loop(0, 250, off, 0)
            pltpu.sync_copy(ones_v, dego_s.at[dsst], add=True)
            pltpu.sync_copy(ones_v, degi_s.at[ddst], add=True)

    plsc.subcore_barrier()
    pltpu.sync_copy(cntloc, cnt.at[pl.ds(wid * 48, 48)])
    pltpu.sync_copy(dego_s.at[pl.ds(s * _DEGT, _DEGT)], zbuf)
    pltpu.sync_copy(zbuf, dego.at[pl.ds(c * _DEGN + s * _DEGT, _DEGT)])
    pltpu.sync_copy(degi_s.at[pl.ds(s * _DEGT, _DEGT)], zbuf)
    pltpu.sync_copy(zbuf, degi.at[pl.ds(c * _DEGN + s * _DEGT, _DEGT)])


_prep_call = functools.partial(
    pl.kernel, _prep_body, mesh=_mesh,
    compiler_params=pltpu.CompilerParams(needs_layout_passes=False),
    out_type=[jax.ShapeDtypeStruct((9 * _NT * _NB * _CAP,), _i32),
              jax.ShapeDtypeStruct((9 * _NT * _NB * _CAP,), _i32),
              jax.ShapeDtypeStruct((_NT * 48,), _i32),
              jax.ShapeDtypeStruct((2 * _DEGN,), _f32),
              jax.ShapeDtypeStruct((2 * _DEGN,), _f32)],
    scratch_types=[pltpu.VMEM((_EPT,), _i32),
                   pltpu.VMEM((_EPT,), _i32)]
    + [pltpu.VMEM((_CAP + 16,), _i32)] * 6
    + [pltpu.VMEM((48,), _i32),
                   pltpu.VMEM((4000,), _f32),
                   pltpu.VMEM((4000,), _i32),
                   pltpu.VMEM((4000,), _i32),
                   pltpu.VMEM((_DEGT,), _f32),
                   pltpu.VMEM_SHARED((_DEGN,), _f32),
                   pltpu.VMEM_SHARED((_DEGN,), _f32)],
)()


def _segsum_body(rels, refs):
    n_z = len(rels)
    zs = refs[:n_z]
    (gsrc, gdst, cnt) = refs[n_z:n_z + 3]
    aggs = refs[n_z + 3:2 * n_z + 3]
    rest = refs[2 * n_z + 3:]
    acc, sseg, dseg = rest[0], rest[1], rest[2]
    rbufs = rest[3:3 + _D]
    zrow, cntloc = rest[3 + _D], rest[4 + _D]
    sems = rest[5 + _D:5 + 2 * _D]
    c = lax.axis_index("c")
    s = lax.axis_index("s")

    def fill(i, _):
        def fl(j, _):
            zrow[i, pl.ds(j * 16, 16)] = jnp.zeros((16,), _f32)
            return 0
        lax.fori_loop(0, 8, fl, 0)
        return 0
    lax.fori_loop(0, 64, fill, 0)
    pltpu.sync_copy(cnt.at[pl.ds(2 * s * 48, 96)], cntloc)

    for task in range(n_z * _NB):
        ri, b = task // _NB, task % _NB
        r = rels[ri]
        owner = task % 2
        zref = zs[ri]
        aref = aggs[ri]

        @pl.when(c == owner)
        def _task():
            for j in range(12):
                pltpu.sync_copy(zrow, acc.at[pl.ds(s * 768 + j * 64, 64)])
            plsc.subcore_barrier()
            for seg in range(2):
                t = 2 * s + seg
                gbase = ((r * _NT + t) * _NB + b) * _CAP
                pltpu.sync_copy(gsrc.at[pl.ds(gbase, _CAP)], sseg)
                pltpu.sync_copy(gdst.at[pl.ds(gbase, _CAP)], dseg)
                lane = r * _NB + b
                vec = cntloc[pl.ds(seg * 48 + (lane // 16) * 16, 16)]
                iot = lax.iota(_i32, 16)
                cval = jnp.max(jnp.where(iot == (lane % 16), vec, 0))
                nch = (cval + (_K - 1)) >> _KSH

                for p in range(_D - 1):
                    @pl.when(p < nch)
                    def _prime():
                        pltpu.async_copy(zref.at[sseg.at[pl.ds(p * _K, _K)]],
                                         rbufs[p], sems[p])

                def chunk(ci, _):
                    for par in range(_D):
                        rb, sm = rbufs[par], sems[par]

                        @pl.when(ci % _D == par)
                        def _():
                            nxt = ci + (_D - 1)

                            @pl.when(nxt < nch)
                            def _start_next():
                                pltpu.async_copy(
                                    zref.at[sseg.at[pl.ds(nxt * _K, _K)]],
                                    rbufs[(par + _D - 1) % _D],
                                    sems[(par + _D - 1) % _D])
                            pltpu.make_async_copy(
                                zref.at[pl.ds(0, _K)], rb, sm).wait()
                            pltpu.sync_copy(
                                rb, acc.at[dseg.at[pl.ds(ci * _K, _K)]],
                                add=True)
                    return 0
                lax.fori_loop(0, nch, chunk, 0)
            plsc.subcore_barrier()
            pltpu.sync_copy(acc.at[pl.ds(s * 768, 768)],
                            aref.at[pl.ds(b * _R + s * 768, 768)])


def _segsum_call(rels, zs, gsrc, gdst, cnt):
    def body(*refs):
        _segsum_body(rels, refs)

    call = functools.partial(
        pl.kernel, body, mesh=_mesh,
        compiler_params=pltpu.CompilerParams(needs_layout_passes=False),
        out_type=[jax.ShapeDtypeStruct((_NPAD, 128), _f32)] * len(rels),
        scratch_types=[pltpu.VMEM_SHARED((_ACC, 128), _f32),
                       pltpu.VMEM((_CAP,), _i32),
                       pltpu.VMEM((_CAP,), _i32)]
        + [pltpu.VMEM((_K, 128), _f32)] * _D
        + [pltpu.VMEM((64, 128), _f32),
           pltpu.VMEM((96,), _i32)]
        + [pltpu.SemaphoreType.DMA] * _D,
    )()
    return call(*zs, gsrc, gdst, cnt)


def _scale(deg_blk):
    return jax.lax.rsqrt(jnp.maximum(deg_blk, 1.0))


def _stage_out_body(x_ref, d0, d1, d2, w0, w1, w2, z0, z1, z2):
    x = x_ref[...]
    for dref, wref, zref in ((d0, w0, z0), (d1, w1, z1), (d2, w2, z2)):
        s = _scale(dref[...])
        zref[...] = jnp.dot(x * s[:, None], wref[...],
                            preferred_element_type=jnp.float32)


def _stage_out(x, degs, Ws):
    n = x.shape[0]
    nblk = pl.cdiv(n, _BLK)
    row = pl.BlockSpec((_BLK, 128), lambda j: (j, 0))
    vec = pl.BlockSpec((_BLK,), lambda j: (j,))
    wsp = pl.BlockSpec((128, 128), lambda j: (0, 0))
    return pl.pallas_call(
        _stage_out_body,
        grid=(nblk,),
        in_specs=[row, vec, vec, vec, wsp, wsp, wsp],
        out_specs=[row, row, row],
        out_shape=[jax.ShapeDtypeStruct((n, 128), jnp.float32)] * 3,
    )(x, degs[0], degs[1], degs[2], Ws[0], Ws[1], Ws[2])


def _stage_mid_body(n_in, refs):
    aggs = refs[:n_in]
    dins = refs[n_in:2 * n_in]
    bsum = refs[2 * n_in]
    douts = refs[2 * n_in + 1:2 * n_in + 4]
    ws = refs[2 * n_in + 4:2 * n_in + 7]
    zs = refs[2 * n_in + 7:]
    h = bsum[...]
    for a, d in zip(aggs, dins):
        h = h + a[...] * _scale(d[...])[:, None]
    h = jnp.maximum(h, 0.0)
    for d, w, z in zip(douts, ws, zs):
        s = _scale(d[...])
        z[...] = jnp.dot(h * s[:, None], w[...],
                         preferred_element_type=jnp.float32)


def _stage_mid(aggs, dins, bsum, douts, Ws, n):
    n_in = len(aggs)
    nblk = pl.cdiv(n, _BLK)
    row = pl.BlockSpec((_BLK, 128), lambda j: (j, 0))
    vec = pl.BlockSpec((_BLK,), lambda j: (j,))
    wsp = pl.BlockSpec((128, 128), lambda j: (0, 0))
    bsp = pl.BlockSpec((1, 128), lambda j: (0, 0))

    def body(*refs):
        _stage_mid_body(n_in, refs)

    return pl.pallas_call(
        body,
        grid=(nblk,),
        in_specs=[row] * n_in + [vec] * n_in + [bsp] + [vec] * 3 + [wsp] * 3,
        out_specs=[row] * 3,
        out_shape=[jax.ShapeDtypeStruct((n, 128), jnp.float32)] * 3,
    )(*aggs, *dins, bsum.reshape(1, 128), *douts, *Ws)


def _stage_final_body(n_in, refs):
    aggs = refs[:n_in]
    dins = refs[n_in:2 * n_in]
    bsum = refs[2 * n_in]
    wc = refs[2 * n_in + 1]
    bc = refs[2 * n_in + 2]
    out = refs[2 * n_in + 3]
    h = bsum[...]
    for a, d in zip(aggs, dins):
        h = h + a[...] * _scale(d[...])[:, None]
    h = jnp.maximum(h, 0.0)
    out[...] = jnp.dot(h, wc[...], preferred_element_type=jnp.float32) + bc[...]


def _stage_final(aggs, dins, bsum, Wc, bc, n):
    n_in = len(aggs)
    nblk = pl.cdiv(n, _BLK)
    row = pl.BlockSpec((_BLK, 128), lambda j: (j, 0))
    vec = pl.BlockSpec((_BLK,), lambda j: (j,))
    bsp = pl.BlockSpec((1, 128), lambda j: (0, 0))

    def body(*refs):
        _stage_final_body(n_in, refs)

    return pl.pallas_call(
        body,
        grid=(nblk,),
        in_specs=[row] * n_in + [vec] * n_in + [bsp]
        + [pl.BlockSpec((128, 16), lambda j: (0, 0)),
           pl.BlockSpec((1, 16), lambda j: (0, 0))],
        out_specs=pl.BlockSpec((_BLK, 16), lambda j: (j, 0)),
        out_shape=jax.ShapeDtypeStruct((n, 16), jnp.float32),
    )(*aggs, *dins, bsum.reshape(1, 128), Wc, bc.reshape(1, 16))


def kernel(x_assmpt, x_non_assmpt, x_rule, e0, e1, e2, e3, e4, e5, e6, e7, e8,
           W1, b1, W2, b2, Wc, bc):
    xs = {"assmpt": x_assmpt, "non_assmpt": x_non_assmpt, "rule": x_rule}
    edges = [e.astype(jnp.int32) for e in (e0, e1, e2, e3, e4, e5, e6, e7, e8)]

    gsrc, gdst, cnt, dego, degi = _prep_call(
        *[e[0] for e in edges], *[e[1] for e in edges])
    dego = dego.reshape(2, _DEGN)
    degi = degi.reshape(2, _DEGN)
    deg_out = [dego[i % 2, i * _DS:i * _DS + _SIZES[s]]
               for i, (s, _) in enumerate(_RELS)]
    deg_in = [degi[i % 2, i * _DS:i * _DS + _SIZES[d]]
              for i, (_, d) in enumerate(_RELS)]
    bsum1 = {t: functools.reduce(jnp.add, (b1[i] for i in _IN_RELS[t])) for t in _TYPES}
    bsum2 = {t: functools.reduce(jnp.add, (b2[i] for i in _IN_RELS[t])) for t in _TYPES}

    z1 = [None] * 9
    for t in _TYPES:
        rels = _OUT_RELS[t]
        zt = _stage_out(xs[t], [deg_out[i] for i in rels], [W1[i] for i in rels])
        for i, z in zip(rels, zt):
            z1[i] = z
    agg1 = {t: _segsum_call(tuple(_IN_RELS[t]), [z1[i] for i in _IN_RELS[t]],
                            gsrc, gdst, cnt) for t in _TYPES}

    z2 = [None] * 9
    for t in _TYPES:
        irels = _IN_RELS[t]
        orels = _OUT_RELS[t]
        zt = _stage_mid(agg1[t], [deg_in[i] for i in irels],
                        bsum1[t], [deg_out[j] for j in orels],
                        [W2[j] for j in orels], _SIZES[t])
        for j, z in zip(orels, zt):
            z2[j] = z
    agg2 = {t: _segsum_call(tuple(_IN_RELS[t]), [z2[i] for i in _IN_RELS[t]],
                            gsrc, gdst, cnt) for t in _TYPES}

    outs = {}
    for t in _TYPES:
        irels = _IN_RELS[t]
        outs[t] = _stage_final(agg2[t],
                               [deg_in[i] for i in irels], bsum2[t], Wc, bc,
                               _SIZES[t])
    return (outs["assmpt"], outs["non_assmpt"], outs["rule"])
